# baseline (speedup 1.0000x reference)
import jax
import jax.numpy as jnp
from jax import lax
from jax.experimental import pallas as pl
from jax.experimental.pallas import tpu as pltpu

D = 1024
VQ = 256
M = 1024
TILE_M = 1024


def _dot(a, b, prec):
    return lax.dot_general(a, b, (((1,), (0,)), ((), ())), precision=prec,
                           preferred_element_type=jnp.float32)


def _dot_t(a, b, prec):
    return lax.dot_general(a, b, (((1,), (1,)), ((), ())), precision=prec,
                           preferred_element_type=jnp.float32)


def _vq_kernel(xrec_ref, p_ref, cb_ref, out_ref, cbn_ref):
    dp = lax.Precision.DEFAULT

    @pl.when(pl.program_id(0) == 0)
    def _normalize_codebook():
        cb = cb_ref[...]
        cbn_ref[...] = cb / jnp.sqrt(jnp.sum(cb * cb, axis=-1, keepdims=True))

    xrec = xrec_ref[...]
    feat = _dot(xrec, p_ref[...], dp)
    feat = feat / jnp.sqrt(jnp.sum(feat * feat, axis=-1, keepdims=True))
    sim = _dot_t(feat, cbn_ref[...], dp)
    mx = jnp.max(sim, axis=-1, keepdims=True)
    lane = lax.broadcasted_iota(jnp.int32, sim.shape, 1)
    idx = jnp.min(jnp.where(sim == mx, lane, jnp.int32(2**30)), axis=-1)
    out_ref[...] = idx[:, None]


@jax.jit
def kernel(x, projector, codebook):
    B, T, _ = x.shape
    ntok = B * T

    linear = jnp.arange(D // 2 + 1, dtype=x.dtype)
    x_fft = jnp.fft.rfft(x.reshape(128, ntok // 128, D), axis=-1)
    magnitude = jnp.abs(x_fft)
    phase = jnp.angle(x_fft)
    phase = phase - phase[:, :, 1:2] * linear
    x_recon = magnitude * jnp.exp(1j * phase)
    xrec2 = jnp.fft.irfft(x_recon, axis=-1).reshape(ntok, D)
    cb = codebook.reshape(M, VQ)

    grid = (ntok // TILE_M,)
    out = pl.pallas_call(
        _vq_kernel,
        grid=grid,
        in_specs=[
            pl.BlockSpec((TILE_M, D), lambda i: (i, 0)),
            pl.BlockSpec((D, VQ), lambda i: (0, 0)),
            pl.BlockSpec((M, VQ), lambda i: (0, 0)),
        ],
        out_specs=pl.BlockSpec((TILE_M, 1), lambda i: (i, 0)),
        out_shape=jax.ShapeDtypeStruct((ntok, 1), jnp.int32),
        scratch_shapes=[pltpu.VMEM((M, VQ), jnp.float32)],
    )(xrec2, projector, cb)
    return out.reshape(B, T, 1, 1)


# speedup vs baseline: 1.0358x; 1.0358x over previous
import jax
import jax.numpy as jnp
from jax import lax
from jax.experimental import pallas as pl
from jax.experimental.pallas import tpu as pltpu

D = 1024
VQ = 256
M = 1024
TILE_M = 1024


def _dot(a, b, prec):
    return lax.dot_general(a, b, (((1,), (0,)), ((), ())), precision=prec,
                           preferred_element_type=jnp.float32)


def _dot_t(a, b, prec):
    return lax.dot_general(a, b, (((1,), (1,)), ((), ())), precision=prec,
                           preferred_element_type=jnp.float32)


def _vq_kernel(xrec_ref, p_ref, cb_ref, out_ref, cbn_ref):
    dp = lax.Precision.DEFAULT

    @pl.when(pl.program_id(0) == 0)
    def _normalize_codebook():
        cb = cb_ref[...]
        cbn_ref[...] = cb / jnp.sqrt(jnp.sum(cb * cb, axis=-1, keepdims=True))

    xrec = xrec_ref[...]
    feat = _dot(xrec, p_ref[...], dp)
    feat = feat / jnp.sqrt(jnp.sum(feat * feat, axis=-1, keepdims=True))
    sim = _dot_t(feat, cbn_ref[...], dp)
    mx = jnp.max(sim, axis=-1, keepdims=True)
    lane = lax.broadcasted_iota(jnp.int32, sim.shape, 1)
    idx = jnp.min(jnp.where(sim == mx, lane, jnp.int32(2**30)), axis=-1)
    out_ref[...] = idx[:, None]


@jax.jit
def kernel(x, projector, codebook):
    B, T, _ = x.shape
    ntok = B * T

    linear = jnp.arange(D // 2 + 1, dtype=x.dtype)
    x_fft = jnp.fft.rfft(x.reshape(64, ntok // 64, D), axis=-1)
    magnitude = jnp.abs(x_fft)
    phase = jnp.angle(x_fft)
    phase = phase - phase[:, :, 1:2] * linear
    x_recon = magnitude * jnp.exp(1j * phase)
    xrec2 = jnp.fft.irfft(x_recon, axis=-1).reshape(ntok, D)
    cb = codebook.reshape(M, VQ)

    grid = (ntok // TILE_M,)
    out = pl.pallas_call(
        _vq_kernel,
        grid=grid,
        in_specs=[
            pl.BlockSpec((TILE_M, D), lambda i: (i, 0)),
            pl.BlockSpec((D, VQ), lambda i: (0, 0)),
            pl.BlockSpec((M, VQ), lambda i: (0, 0)),
        ],
        out_specs=pl.BlockSpec((TILE_M, 1), lambda i: (i, 0)),
        out_shape=jax.ShapeDtypeStruct((ntok, 1), jnp.int32),
        scratch_shapes=[pltpu.VMEM((M, VQ), jnp.float32)],
    )(xrec2, projector, cb)
    return out.reshape(B, T, 1, 1)
